# baseline (speedup 1.0000x reference)
import functools

import jax
import jax.numpy as jnp
from jax import lax
from jax.experimental import pallas as pl
from jax.experimental.pallas import tpu as pltpu
from jax.experimental.pallas import tpu_sc as plsc

N_NODES = 10000
E = 320000
D = 128
DE = 16
L = 16
NC, NS = 2, 16
NW = NC * NS
C = 32
NCHUNKS = E // C
BASE_CH = NCHUNKS // NW
EXTRA = NCHUNKS - BASE_CH * NW


def _scratches():
    scr = []
    for _ in range(4):
        scr += [pltpu.VMEM((C,), jnp.int32),
                pltpu.VMEM((C,), jnp.int32),
                pltpu.VMEM((C,), jnp.float32),
                pltpu.VMEM((C, DE), jnp.float32)]
    for _ in range(2):
        scr += [pltpu.VMEM((C, D), jnp.float32),
                pltpu.VMEM((C, D), jnp.float32)]
    scr += [pltpu.SemaphoreType.DMA] * 4
    scr += [pltpu.SemaphoreType.DMA] * 4
    scr += [pltpu.SemaphoreType.DMA] * 4
    scr += [pltpu.VMEM_SHARED((N_NODES, D), jnp.float32)]
    return scr


@functools.partial(
    pl.kernel,
    out_type=jax.ShapeDtypeStruct((E, DE), jnp.float32),
    mesh=plsc.VectorSubcoreMesh(core_axis_name="c", subcore_axis_name="s"),
    compiler_params=pltpu.CompilerParams(needs_layout_passes=False),
    scratch_types=_scratches(),
)
def _edge_metrics(nodes, edges, active, senders, receivers, out, *scr):
    aux = [scr[4 * s:4 * s + 4] for s in range(4)]
    rows = [scr[16 + 2 * s:16 + 2 * s + 2] for s in range(2)]
    sem_aux = scr[20:24]
    sem_gs = scr[24:26]
    sem_gr = scr[26:28]
    sem_wb = scr[28:32]
    nodes_sh = scr[32]

    wid = lax.axis_index("s") * NC + lax.axis_index("c")
    nch = BASE_CH + jnp.where(wid < EXTRA, 1, 0)

    def cbase(i):
        return (wid + i * NW) * C

    def issue_aux(i, s):
        b = cbase(i)
        sidx, ridx, act, ed = aux[s]
        pltpu.async_copy(senders.at[pl.ds(b, C)], sidx, sem_aux[s])
        pltpu.async_copy(receivers.at[pl.ds(b, C)], ridx, sem_aux[s])
        pltpu.async_copy(active.at[pl.ds(b, C)], act, sem_aux[s])
        pltpu.async_copy(edges.at[pl.ds(b, C)], ed, sem_aux[s])

    def wait_aux(s):
        sidx, ridx, act, ed = aux[s]
        pltpu.make_async_copy(senders.at[pl.ds(0, C)], sidx, sem_aux[s]).wait()
        pltpu.make_async_copy(receivers.at[pl.ds(0, C)], ridx, sem_aux[s]).wait()
        pltpu.make_async_copy(active.at[pl.ds(0, C)], act, sem_aux[s]).wait()
        pltpu.make_async_copy(edges.at[pl.ds(0, C)], ed, sem_aux[s]).wait()

    def issue_gathers(sa, sr):
        sidx, ridx, _, _ = aux[sa]
        srows, rrows = rows[sr]
        pltpu.async_copy(nodes.at[sidx], srows, sem_gs[sr])
        pltpu.async_copy(nodes_sh.at[ridx], rrows, sem_gr[sr])

    def wait_gathers(sa, sr):
        sidx, ridx, _, _ = aux[sa]
        srows, rrows = rows[sr]
        pltpu.make_async_copy(nodes.at[sidx], srows, sem_gs[sr]).wait()
        pltpu.make_async_copy(nodes_sh.at[ridx], rrows, sem_gr[sr]).wait()

    def issue_wb(i, s):
        pltpu.async_copy(aux[s][3], out.at[pl.ds(cbase(i), C)], sem_wb[s])

    def wait_wb(s):
        pltpu.make_async_copy(aux[s][3], out.at[pl.ds(0, C)], sem_wb[s]).wait()

    def compute(sa, sr):
        _, _, act, ed = aux[sa]
        srows, rrows = rows[sr]
        lane = lax.iota(jnp.int32, L)
        zero = jnp.zeros((L,), jnp.float32)

        def gbody(g, carry):
            agrp = act[pl.ds(g * L, L)]

            def ebody(el, res):
                res1, res2 = res
                e = g * L + el
                acc1 = zero
                acc2 = zero
                for j in range(D // L):
                    s = srows[e, pl.ds(j * L, L)]
                    r = rrows[e, pl.ds(j * L, L)]
                    d = s - r
                    acc1 = acc1 + d * d
                    acc2 = acc2 + s * r
                hit = lane == el
                res1 = jnp.where(hit, jnp.sum(acc1), res1)
                res2 = jnp.where(hit, jnp.sum(acc2), res2)
                return res1, res2

            res1, res2 = lax.fori_loop(0, L, ebody, (zero, zero), unroll=4)
            eidx = lane + g * L
            plsc.store_scatter(ed, [eidx, jnp.zeros((L,), jnp.int32) + (DE - 2)],
                               res1 * agrp)
            plsc.store_scatter(ed, [eidx, jnp.zeros((L,), jnp.int32) + (DE - 1)],
                               res2 * agrp)
            return carry

        lax.fori_loop(0, C // L, gbody, 0)

    @pl.when(lax.axis_index("s") == 0)
    def _():
        pltpu.sync_copy(nodes, nodes_sh)
    plsc.subcore_barrier()

    issue_aux(0, 0)
    issue_aux(1, 1)
    wait_aux(0)
    issue_gathers(0, 0)

    def quad_body(q, carry):
        for b in range(4):
            i = 4 * q + b
            s4 = b
            s2 = b % 2

            @pl.when(i + 1 < nch)
            def _():
                wait_aux((b + 1) % 4)
                issue_gathers((b + 1) % 4, (b + 1) % 2)

            @pl.when(i < nch)
            def _():
                wait_gathers(s4, s2)

                @pl.when(i + 2 < nch)
                def _():
                    @pl.when(i >= 2)
                    def _():
                        wait_wb((b + 2) % 4)
                    issue_aux(i + 2, (b + 2) % 4)

                compute(s4, s2)
                issue_wb(i, s4)

        return carry

    nquad = (nch + 3) // 4
    lax.fori_loop(0, nquad, quad_body, 0)

    for s in range(4):
        wait_wb(s)


def kernel(nodes, edges, active_edges, senders, receivers):
    return _edge_metrics(nodes, edges, active_edges,
                         senders.astype(jnp.int32),
                         receivers.astype(jnp.int32))


# speedup vs baseline: 1.2172x; 1.2172x over previous
import functools

import jax
import jax.numpy as jnp
from jax import lax
from jax.experimental import pallas as pl
from jax.experimental.pallas import tpu as pltpu
from jax.experimental.pallas import tpu_sc as plsc

N_NODES = 10000
E = 320000
D = 128
DE = 16
L = 16
NC, NS = 2, 16
NW = NC * NS
C = 128
NCHUNKS = E // C
BASE_CH = NCHUNKS // NW
EXTRA = NCHUNKS - BASE_CH * NW
AR = 3
RR = 2
UN = 6


def _scratches():
    scr = []
    for _ in range(AR):
        scr += [pltpu.VMEM((C,), jnp.int32),
                pltpu.VMEM((C,), jnp.int32),
                pltpu.VMEM((C,), jnp.float32),
                pltpu.VMEM((C, DE), jnp.float32)]
    for _ in range(RR):
        scr += [pltpu.VMEM((C, D), jnp.float32),
                pltpu.VMEM((C, D), jnp.float32)]
    scr += [pltpu.SemaphoreType.DMA] * AR
    scr += [pltpu.SemaphoreType.DMA] * RR
    scr += [pltpu.SemaphoreType.DMA] * AR
    return scr


@functools.partial(
    pl.kernel,
    out_type=jax.ShapeDtypeStruct((E, DE), jnp.float32),
    mesh=plsc.VectorSubcoreMesh(core_axis_name="c", subcore_axis_name="s"),
    compiler_params=pltpu.CompilerParams(needs_layout_passes=False),
    scratch_types=_scratches(),
)
def _edge_metrics(nodes, edges, active, senders, receivers, out, *scr):
    na = 4 * AR
    aux = [scr[4 * s:4 * s + 4] for s in range(AR)]
    rows = [scr[na + 2 * s:na + 2 * s + 2] for s in range(RR)]
    ns = na + 2 * RR
    sem_aux = scr[ns:ns + AR]
    sem_g = scr[ns + AR:ns + AR + RR]
    sem_wb = scr[ns + AR + RR:ns + 2 * AR + RR]

    wid = lax.axis_index("s") * NC + lax.axis_index("c")
    nch = BASE_CH + jnp.where(wid < EXTRA, 1, 0)

    def cbase(i):
        return (wid + i * NW) * C

    def issue_aux(i, s):
        b = cbase(i)
        sidx, ridx, act, ed = aux[s]
        pltpu.async_copy(senders.at[pl.ds(b, C)], sidx, sem_aux[s])
        pltpu.async_copy(receivers.at[pl.ds(b, C)], ridx, sem_aux[s])
        pltpu.async_copy(active.at[pl.ds(b, C)], act, sem_aux[s])
        pltpu.async_copy(edges.at[pl.ds(b, C)], ed, sem_aux[s])

    def wait_aux(s):
        sidx, ridx, act, ed = aux[s]
        pltpu.make_async_copy(senders.at[pl.ds(0, C)], sidx, sem_aux[s]).wait()
        pltpu.make_async_copy(receivers.at[pl.ds(0, C)], ridx, sem_aux[s]).wait()
        pltpu.make_async_copy(active.at[pl.ds(0, C)], act, sem_aux[s]).wait()
        pltpu.make_async_copy(edges.at[pl.ds(0, C)], ed, sem_aux[s]).wait()

    def issue_gathers(sa, sr):
        sidx, ridx, _, _ = aux[sa]
        srows, rrows = rows[sr]
        pltpu.async_copy(nodes.at[sidx], srows, sem_g[sr])
        pltpu.async_copy(nodes.at[ridx], rrows, sem_g[sr])

    def wait_gathers(sa, sr):
        sidx, ridx, _, _ = aux[sa]
        srows, rrows = rows[sr]
        pltpu.make_async_copy(nodes.at[sidx], srows, sem_g[sr]).wait()
        pltpu.make_async_copy(nodes.at[ridx], rrows, sem_g[sr]).wait()

    def issue_wb(i, s):
        pltpu.async_copy(aux[s][3], out.at[pl.ds(cbase(i), C)], sem_wb[s])

    def wait_wb(s):
        pltpu.make_async_copy(aux[s][3], out.at[pl.ds(0, C)], sem_wb[s]).wait()

    def compute(sa, sr):
        _, _, act, ed = aux[sa]
        srows, rrows = rows[sr]
        lane = lax.iota(jnp.int32, L)
        zero = jnp.zeros((L,), jnp.float32)

        def gbody(g, carry):
            agrp = act[pl.ds(g * L, L)]

            def ebody(el, res):
                res1, res2 = res
                e = g * L + el
                acc1 = zero
                acc2 = zero
                for j in range(D // L):
                    s = srows[e, pl.ds(j * L, L)]
                    r = rrows[e, pl.ds(j * L, L)]
                    d = s - r
                    acc1 = acc1 + d * d
                    acc2 = acc2 + s * r
                hit = lane == el
                res1 = jnp.where(hit, jnp.sum(acc1), res1)
                res2 = jnp.where(hit, jnp.sum(acc2), res2)
                return res1, res2

            res1, res2 = lax.fori_loop(0, L, ebody, (zero, zero), unroll=4)
            eidx = lane + g * L
            plsc.store_scatter(ed, [eidx, jnp.zeros((L,), jnp.int32) + (DE - 2)],
                               res1 * agrp)
            plsc.store_scatter(ed, [eidx, jnp.zeros((L,), jnp.int32) + (DE - 1)],
                               res2 * agrp)
            return carry

        lax.fori_loop(0, C // L, gbody, 0)

    issue_aux(0, 0)
    issue_aux(1, 1)
    wait_aux(0)
    issue_gathers(0, 0)

    def block_body(q, carry):
        for b in range(UN):
            i = UN * q + b
            sa = b % AR
            sr = b % RR

            @pl.when(i + 1 < nch)
            def _():
                wait_aux((b + 1) % AR)
                issue_gathers((b + 1) % AR, (b + 1) % RR)

            @pl.when(i < nch)
            def _():
                wait_gathers(sa, sr)

                @pl.when(i + 2 < nch)
                def _():
                    @pl.when(i >= 1)
                    def _():
                        wait_wb((b + 2) % AR)
                    issue_aux(i + 2, (b + 2) % AR)

                compute(sa, sr)
                issue_wb(i, sa)

        return carry

    nblk = (nch + UN - 1) // UN
    lax.fori_loop(0, nblk, block_body, 0)

    for s in range(AR):
        wait_wb(s)


def kernel(nodes, edges, active_edges, senders, receivers):
    return _edge_metrics(nodes, edges, active_edges,
                         senders.astype(jnp.int32),
                         receivers.astype(jnp.int32))
